# baseline (speedup 1.0000x reference)
import functools

import jax
import jax.numpy as jnp
from jax.experimental import pallas as pl
from jax.experimental.pallas import tpu as pltpu

B = 4096
N = 1000
BM = 128

_CLS_COEF = 0.002
_OFF_COEF = 0.004


def _loss_kernel(cls_ref, y_ref, po_ref, oe_ref, out_ref):
    i = pl.program_id(0)

    z = cls_ref[...]
    yv = y_ref[...]
    po = po_ref[...]
    oe = oe_ref[...]

    parity = jax.lax.broadcasted_iota(jnp.int32, (BM, 2 * N), 1) % 2
    is_odd = parity == 1

    z_next = pltpu.roll(z, 2 * N - 1, 1)
    cmp = (z_next > z).astype(jnp.float32)
    cmp_r = pltpu.roll(cmp, 1, 1)
    pos = jnp.where(is_odd, cmp_r, cmp)

    off_term = jnp.abs(po - oe * pos)

    y2 = pltpu.repeat(yv, 2, 1)
    ls = -(jnp.maximum(-z, 0.0) + jnp.log1p(jnp.exp(-jnp.abs(z))))
    bce = (1.0 - y2) * z - (1.0 + 9.0 * y2) * ls
    bce = jnp.where(is_odd, bce, 0.0)

    part = _CLS_COEF * jnp.sum(bce) + _OFF_COEF * jnp.sum(off_term)
    part = jnp.full((1, 1), part, dtype=jnp.float32)

    @pl.when(i == 0)
    def _init():
        out_ref[...] = jnp.zeros((1, 1), jnp.float32)

    out_ref[...] += part


@functools.partial(jax.jit, static_argnums=())
def _tnt_loss(cls2, y, po2, oe2):
    grid = (B // BM,)
    out = pl.pallas_call(
        _loss_kernel,
        grid=grid,
        in_specs=[
            pl.BlockSpec((BM, 2 * N), lambda i: (i, 0)),
            pl.BlockSpec((BM, N), lambda i: (i, 0)),
            pl.BlockSpec((BM, 2 * N), lambda i: (i, 0)),
            pl.BlockSpec((BM, 2 * N), lambda i: (i, 0)),
        ],
        out_specs=pl.BlockSpec((1, 1), lambda i: (0, 0)),
        out_shape=jax.ShapeDtypeStruct((1, 1), jnp.float32),
    )(cls2, y, po2, oe2)
    return out[0, 0]


def kernel(pred_target_prob, pred_offset, pred_RCNN_cls, gt_target_prob,
           gt_candidate, gt_offset_each, gt_target_candidate_lens):
    cls2 = pred_RCNN_cls.reshape(B, 2 * N)
    po2 = pred_offset.reshape(B, 2 * N)
    oe2 = gt_offset_each.reshape(B, 2 * N)
    return _tnt_loss(cls2, gt_target_prob, po2, oe2)


# speedup vs baseline: 5.6215x; 5.6215x over previous
import jax
import jax.numpy as jnp
from jax.experimental import pallas as pl
from jax.experimental.pallas import tpu as pltpu

B = 4096
N = 1000
BM = 128
GRID = B // BM

_CLS_COEF = 0.002
_OFF_COEF = 0.004


def _loss_kernel(c0_ref, c1_ref, y_ref, p0_ref, p1_ref, e0_ref, e1_ref,
                 out_ref):
    i = pl.program_id(0)

    c0 = c0_ref[...]
    c1 = c1_ref[...]
    yv = y_ref[...]

    pos = (c1 > c0).astype(jnp.float32)
    off = (jnp.abs(p0_ref[...] - e0_ref[...] * pos)
           + jnp.abs(p1_ref[...] - e1_ref[...] * pos))

    ls = -(jnp.maximum(-c1, 0.0) + jnp.log1p(jnp.exp(-jnp.abs(c1))))
    bce = (1.0 - yv) * c1 - (1.0 + 9.0 * yv) * ls

    part = _CLS_COEF * jnp.sum(bce) + _OFF_COEF * jnp.sum(off)
    part = jnp.full((1, 1), part, dtype=jnp.float32)

    @pl.when(i == 0)
    def _init():
        out_ref[...] = jnp.zeros((1, 1), jnp.float32)

    out_ref[...] += part


def _tnt_loss(c0, c1, y, p0, p1, e0, e1):
    spec = pl.BlockSpec((BM, N), lambda i: (i, 0))
    out = pl.pallas_call(
        _loss_kernel,
        grid=(GRID,),
        in_specs=[spec] * 7,
        out_specs=pl.BlockSpec((1, 1), lambda i: (0, 0)),
        out_shape=jax.ShapeDtypeStruct((1, 1), jnp.float32),
        compiler_params=pltpu.CompilerParams(
            dimension_semantics=("arbitrary",),
            allow_input_fusion=[True] * 7,
        ),
    )(c0, c1, y, p0, p1, e0, e1)
    return out[0, 0]


def kernel(pred_target_prob, pred_offset, pred_RCNN_cls, gt_target_prob,
           gt_candidate, gt_offset_each, gt_target_candidate_lens):
    oe3 = gt_offset_each.reshape(B, N, 2)
    c0 = pred_RCNN_cls[:, :, 0]
    c1 = pred_RCNN_cls[:, :, 1]
    p0 = pred_offset[:, :, 0]
    p1 = pred_offset[:, :, 1]
    e0 = oe3[:, :, 0]
    e1 = oe3[:, :, 1]
    return _tnt_loss(c0, c1, gt_target_prob, p0, p1, e0, e1)
